# baseline (speedup 1.0000x reference)
import functools

import jax
import jax.numpy as jnp
from jax import lax
from jax.experimental import pallas as pl
from jax.experimental.pallas import tpu as pltpu
from jax.experimental.pallas import tpu_sc as plsc

BATCH = 16384
ATOMS = 200
NUM_SPECIES = 8

NC = 2
NS = 16
LANES = 16
NW = NC * NS
ROWS = BATCH // NW
CHR = 256
NCH = ROWS // CHR
GPC = CHR // LANES


def _lookup(tab_reg, sv):
    return tab_reg.at[sv].get(mode="promise_in_bounds")


def _sc_body(species_hbm, energies_hbm, table_hbm, out_hbm,
             spec_v, en_v, tab_v, acc_v, out_v):
    wid = lax.axis_index("s") * NC + lax.axis_index("c")
    base = wid * ROWS

    lanes = lax.iota(jnp.int32, LANES)
    zero = jnp.where(lanes < 0, 1.0, 0.0).astype(jnp.float32)
    hi_mask = lanes >= 8

    tab_v[...] = zero
    pltpu.sync_copy(table_hbm, tab_v.at[pl.ds(0, NUM_SPECIES)])
    pltpu.sync_copy(energies_hbm.at[pl.ds(base, ROWS)], en_v)

    tab_reg = tab_v[...]
    rot_store = [r * LANES + ((lanes + r) & (LANES - 1))
                 for r in range(LANES)]
    diag_read = [lanes * LANES + ((lanes + j) & (LANES - 1))
                 for j in range(LANES)]

    for ch in range(NCH):
        crow = base + ch * CHR
        pltpu.sync_copy(species_hbm.at[pl.ds(crow, CHR)], spec_v)

        def group_body(g, carry):
            row0 = pl.multiple_of(g * LANES, LANES)
            for rl in range(LANES):
                r = row0 + rl
                acc = zero
                for j in range(12):
                    sv = spec_v[r, pl.ds(16 * j, LANES)]
                    acc = acc + _lookup(tab_reg, sv)
                svt = spec_v[r, pl.ds(ATOMS - LANES, LANES)]
                mid = _lookup(tab_reg, svt)
                acc = acc + jnp.where(hi_mask, mid, zero)
                plsc.store_scatter(acc_v, [rot_store[rl]], acc)
            tot = zero
            for j in range(LANES):
                tot = tot + plsc.load_gather(acc_v, [diag_read[j]])
            ob = pl.multiple_of(ch * CHR + g * LANES, LANES)
            out_v[pl.ds(ob, LANES)] = tot + en_v[pl.ds(ob, LANES)]
            return carry

        lax.fori_loop(0, GPC, group_body, 0)

    pltpu.sync_copy(out_v, out_hbm.at[pl.ds(base, ROWS)])


@functools.partial(jax.jit)
def _sc_shift(species, energies, self_energies):
    mesh = plsc.VectorSubcoreMesh(core_axis_name="c", subcore_axis_name="s")
    f = pl.kernel(
        _sc_body,
        out_type=jax.ShapeDtypeStruct((BATCH,), jnp.float32),
        mesh=mesh,
        compiler_params=pltpu.CompilerParams(needs_layout_passes=False),
        scratch_types=[
            pltpu.VMEM((CHR, ATOMS), jnp.int32),
            pltpu.VMEM((ROWS,), jnp.float32),
            pltpu.VMEM((LANES,), jnp.float32),
            pltpu.VMEM((LANES * LANES,), jnp.float32),
            pltpu.VMEM((ROWS,), jnp.float32),
        ],
    )
    return f(species, energies, self_energies)


def kernel(species, energies, self_energies):
    shifted = _sc_shift(species, energies, self_energies)
    return (species, shifted)


# speedup vs baseline: 1.0840x; 1.0840x over previous
import functools

import numpy as np
import jax
import jax.numpy as jnp
from jax import lax
from jax.experimental import pallas as pl
from jax.experimental.pallas import tpu as pltpu
from jax.experimental.pallas import tpu_sc as plsc

BATCH = 16384
ATOMS = 200
NUM_SPECIES = 8


SC_ROWS = 4096
NC = 2
NS = 16
LANES = 16
NW = NC * NS
WROWS = SC_ROWS // NW
GPW = WROWS // LANES


def _lookup(tab_reg, sv):
    return tab_reg.at[sv].get(mode="promise_in_bounds")


def _sc_body(species_hbm, energies_hbm, table_hbm, out_hbm,
             spec_v, en_v, tab_v, acc_v, out_v):
    wid = lax.axis_index("s") * NC + lax.axis_index("c")
    base = wid * WROWS

    lanes = lax.iota(jnp.int32, LANES)
    zero = jnp.where(lanes < 0, 1.0, 0.0).astype(jnp.float32)
    hi_mask = lanes >= 8

    tab_v[...] = zero
    pltpu.sync_copy(table_hbm, tab_v.at[pl.ds(0, NUM_SPECIES)])
    pltpu.sync_copy(energies_hbm.at[pl.ds(base, WROWS)], en_v)
    pltpu.sync_copy(species_hbm.at[pl.ds(base, WROWS)], spec_v)

    tab_reg = tab_v[...]
    rot_store = [r * LANES + ((lanes + r) & (LANES - 1))
                 for r in range(LANES)]
    diag_read = [lanes * LANES + ((lanes + j) & (LANES - 1))
                 for j in range(LANES)]

    def group_body(g, carry):
        row0 = pl.multiple_of(g * LANES, LANES)
        for rl in range(LANES):
            r = row0 + rl
            acc = zero
            for j in range(12):
                sv = spec_v[r, pl.ds(16 * j, LANES)]
                acc = acc + _lookup(tab_reg, sv)
            svt = spec_v[r, pl.ds(ATOMS - LANES, LANES)]
            mid = _lookup(tab_reg, svt)
            acc = acc + jnp.where(hi_mask, mid, zero)
            plsc.store_scatter(acc_v, [rot_store[rl]], acc)
        tot = zero
        for j in range(LANES):
            tot = tot + plsc.load_gather(acc_v, [diag_read[j]])
        out_v[pl.ds(row0, LANES)] = tot + en_v[pl.ds(row0, LANES)]
        return carry

    lax.fori_loop(0, GPW, group_body, 0)
    pltpu.sync_copy(out_v, out_hbm.at[pl.ds(base, WROWS)])


def _sc_shift(species, energies, self_energies):
    mesh = plsc.VectorSubcoreMesh(core_axis_name="c", subcore_axis_name="s")
    f = pl.kernel(
        _sc_body,
        out_type=jax.ShapeDtypeStruct((SC_ROWS,), jnp.float32),
        mesh=mesh,
        compiler_params=pltpu.CompilerParams(needs_layout_passes=False),
        scratch_types=[
            pltpu.VMEM((WROWS, ATOMS), jnp.int32),
            pltpu.VMEM((WROWS,), jnp.float32),
            pltpu.VMEM((LANES,), jnp.float32),
            pltpu.VMEM((LANES * LANES,), jnp.float32),
            pltpu.VMEM((WROWS,), jnp.float32),
        ],
    )
    return f(species, energies[:SC_ROWS], self_energies)


TC_ROWS = BATCH - SC_ROWS
CH = 2048
NBUF = 4
NCHUNK = TC_ROWS // CH
OFFC = SC_ROWS // CH

_VINV = np.linalg.inv(
    np.vander(np.arange(NUM_SPECIES), NUM_SPECIES, increasing=True)
    .astype(np.float64))


def _tc_body(tab_ref, spec_hbm, out_ref, buf, sems):
    i = pl.program_id(0)

    def start(j):
        pltpu.make_async_copy(
            spec_hbm.at[pl.ds(SC_ROWS + j * CH, CH), :],
            buf.at[jax.lax.rem(j, NBUF)],
            sems.at[jax.lax.rem(j, NBUF)],
        ).start()

    @pl.when(i == 0)
    def _prologue():
        for j in range(min(NBUF - 1, NCHUNK)):
            start(j)

    @pl.when(i + NBUF - 1 < NCHUNK)
    def _ahead():
        start(i + NBUF - 1)

    pltpu.make_async_copy(
        spec_hbm.at[pl.ds(SC_ROWS + i * CH, CH), :],
        buf.at[jax.lax.rem(i, NBUF)],
        sems.at[jax.lax.rem(i, NBUF)],
    ).wait()

    coef = [None] * NUM_SPECIES
    for m in range(NUM_SPECIES):
        c = None
        for k in range(NUM_SPECIES):
            w = float(_VINV[m, k])
            if w == 0.0:
                continue
            term = w * tab_ref[k]
            c = term if c is None else c + term
        coef[m] = c

    xf = buf[jax.lax.rem(i, NBUF)].astype(jnp.float32)
    val = jnp.full(xf.shape, 0.0, jnp.float32) + coef[NUM_SPECIES - 1]
    for m in range(NUM_SPECIES - 2, -1, -1):
        val = val * xf + coef[m]
    ones = jnp.ones((ATOMS, 8), jnp.float32)
    out_ref[...] = jax.lax.dot_general(
        val, ones, (((1,), (0,)), ((), ())),
        preferred_element_type=jnp.float32)


def _tc_shift(species, energies, self_energies):
    sae8 = pl.pallas_call(
        _tc_body,
        grid=(NCHUNK,),
        in_specs=[
            pl.BlockSpec(memory_space=pltpu.SMEM),
            pl.BlockSpec(memory_space=pl.ANY),
        ],
        out_specs=pl.BlockSpec((CH, 8), lambda i: (i, 0)),
        out_shape=jax.ShapeDtypeStruct((TC_ROWS, 8), jnp.float32),
        scratch_shapes=[
            pltpu.VMEM((NBUF, CH, ATOMS), jnp.int32),
            pltpu.SemaphoreType.DMA((NBUF,)),
        ],
        compiler_params=pltpu.CompilerParams(
            dimension_semantics=("arbitrary",)),
    )(self_energies, species)
    return energies[SC_ROWS:] + sae8[:, 0]


@functools.partial(jax.jit)
def _shift(species, energies, self_energies):
    sc_part = _sc_shift(species, energies, self_energies)
    tc_part = _tc_shift(species, energies, self_energies)
    return jnp.concatenate([sc_part, tc_part])


def kernel(species, energies, self_energies):
    shifted = _shift(species, energies, self_energies)
    return (species, shifted)
